# baseline (speedup 1.0000x reference)
import functools
import math

import jax
import jax.numpy as jnp
from jax import lax
from jax.experimental import pallas as pl
from jax.experimental.pallas import tpu as pltpu
from jax.experimental.pallas import tpu_sc as plsc

NCU = 2
NS = 16
NW = NCU * NS
LN = 16
C = 64
CI = 2048
BLK = 128


def _proj_body(x0, x1, x2, wlin, ws, wqkv, bb, bqkv, base, qkv):
    d = wlin.shape[0]
    acc = jnp.dot(x0[...], wlin[...], preferred_element_type=jnp.float32)
    acc += jnp.dot(x1[...], ws[0], preferred_element_type=jnp.float32)
    acc += jnp.dot(x2[...], ws[1], preferred_element_type=jnp.float32)
    base[...] = acc + bb[...]
    p0 = jnp.dot(x1[...], wqkv[0], preferred_element_type=jnp.float32) + bqkv[0]
    p1 = jnp.dot(x2[...], wqkv[1], preferred_element_type=jnp.float32) + bqkv[1]
    for j in range(3):
        qkv[0, j] = p0[:, j * d:(j + 1) * d]
        qkv[1, j] = p1[:, j * d:(j + 1) * d]


def _post_body(base, agg, out):
    s = agg[...]
    d = base.shape[1]
    r = base[...]
    for h in range(s.shape[0]):
        dd = jnp.sum(s[h][:, d:d + LN], axis=1, keepdims=True) + 1e-16
        r = r + s[h][:, :d] / dd
    out[...] = r


CROWS = CI // 128


def _sc_body(e_true, nch, nsp, d,
             qkv_hbm, ei_hbm, num_out,
             sa, da, sb, db, ria, rib, pend_src, pend_dst,
             q_rows, k_rows, v_rows, exbuf, acc, cnt,
             sem_sa, sem_da, sem_sb, sem_db, sem_q, sem_k, sem_v):
    cid = lax.axis_index("c")
    sid = lax.axis_index("s")
    gid = sid * NCU + cid
    ng = d // LN
    rpt = nsp // NW
    lo = gid * rpt
    hi = lo + rpt
    iota = lax.iota(jnp.int32, LN)
    zero = jnp.zeros((LN,), jnp.float32)
    zrow = iota * 0
    inv = 1.0 / math.sqrt(d)

    def pinit(i, cy):
        pend_src[pl.ds(i * LN, LN)] = zrow
        pend_dst[pl.ds(i * LN, LN)] = zrow + lo
        return cy
    lax.fori_loop(0, C // LN, pinit, 0)

    for hop in range(2):
        q_hbm = qkv_hbm.at[hop, 0]
        k_hbm = qkv_hbm.at[hop, 1]
        v_hbm = qkv_hbm.at[hop, 2]
        src_hbm = ei_hbm.at[hop, 0]
        dst_hbm = ei_hbm.at[hop, 1]

        def zacc(r, cy):
            for g in range(ng + 8):
                acc[r, pl.ds(g * LN, LN)] = zero
            return cy
        lax.fori_loop(0, rpt, zacc, 0)
        cnt[0] = 0

        def flush():
            cq = pltpu.async_copy(q_hbm.at[pend_dst], q_rows, sem_q)
            ck = pltpu.async_copy(k_hbm.at[pend_src], k_rows, sem_k)
            cv = pltpu.async_copy(v_hbm.at[pend_src], v_rows, sem_v)
            cq.wait()
            ck.wait()
            m = cnt[0]

            def dot_g(g, cy):
                gb = g * LN

                def dot_j(j, accs):
                    e = gb + j
                    p = q_rows[e, pl.ds(0, LN)] * k_rows[e, pl.ds(0, LN)]
                    for dg in range(1, ng):
                        p += q_rows[e, pl.ds(dg * LN, LN)] * \
                            k_rows[e, pl.ds(dg * LN, LN)]
                    return jnp.where(iota == j, jnp.sum(p), accs)
                accs = lax.fori_loop(0, LN, dot_j,
                                     jnp.zeros((LN,), jnp.float32))
                ex = jnp.where(gb + iota < m, jnp.exp(accs * inv), 0.0)
                exbuf[pl.ds(gb, LN)] = ex
                return cy
            lax.fori_loop(0, C // LN, dot_g, 0)
            cv.wait()

            def upd_g(g, cy):
                exv = exbuf[pl.ds(g * LN, LN)]
                dlv = pend_dst[pl.ds(g * LN, LN)] - lo
                for j in range(LN):
                    a = exv[j]
                    dl = dlv[j]
                    e = g * LN + j
                    row = zrow + dl
                    for dg in range(ng):
                        vv = v_rows[e, pl.ds(dg * LN, LN)] * a
                        plsc.addupdate_scatter(
                            acc, [row, dg * LN + iota], vv)
                    plsc.addupdate_scatter(
                        acc, [row, d + iota], jnp.where(iota == j, a, 0.0))
                return cy
            lax.fori_loop(0, C // LN, upd_g, 0)
            cnt[0] = 0

        def scan(sbuf, dbuf, base_pos):
            def grp(g, cy):
                r = g // 4
                cc = (g % 4) * 2 * LN

                @pl.when(cnt[0] >= C - 2 * LN)
                def _():
                    flush()

                off = cnt[0]
                for h in range(2):
                    srcg = sbuf[r, pl.ds(cc + h * LN, LN)]
                    dstg = dbuf[r, pl.ds(cc + h * LN, LN)]
                    match = (dstg >= lo) & (dstg < hi)
                    plsc.store_compressed(pend_src.at[pl.ds(off, LN)], srcg,
                                          mask=match)
                    plsc.store_compressed(pend_dst.at[pl.ds(off, LN)], dstg,
                                          mask=match)
                    off = off + plsc.all_reduce_population_count(match)[0]
                cnt[0] = off
                return cy
            lax.fori_loop(0, CI // (2 * LN), grp, 0)

        ria[pl.ds(0, LN)] = iota
        pltpu.async_copy(src_hbm.at[ria], sa, sem_sa)
        pltpu.async_copy(dst_hbm.at[ria], da, sem_da)

        def big(c2, cy):
            pltpu.make_async_copy(src_hbm.at[pl.ds(0, CROWS)], sa, sem_sa).wait()
            pltpu.make_async_copy(dst_hbm.at[pl.ds(0, CROWS)], da, sem_da).wait()
            rib[pl.ds(0, LN)] = (2 * c2 + 1) * CROWS + iota
            pltpu.async_copy(src_hbm.at[rib], sb, sem_sb)
            pltpu.async_copy(dst_hbm.at[rib], db, sem_db)
            scan(sa, da, 2 * c2 * CI)
            pltpu.make_async_copy(src_hbm.at[pl.ds(0, CROWS)], sb, sem_sb).wait()
            pltpu.make_async_copy(dst_hbm.at[pl.ds(0, CROWS)], db, sem_db).wait()
            ria[pl.ds(0, LN)] = (2 * c2 + 2) * CROWS + iota
            pltpu.async_copy(src_hbm.at[ria], sa, sem_sa)
            pltpu.async_copy(dst_hbm.at[ria], da, sem_da)
            scan(sb, db, (2 * c2 + 1) * CI)
            return cy
        lax.fori_loop(0, nch // 2, big, 0)
        pltpu.make_async_copy(src_hbm.at[pl.ds(0, CROWS)], sa, sem_sa).wait()
        pltpu.make_async_copy(dst_hbm.at[pl.ds(0, CROWS)], da, sem_da).wait()
        flush()

        pltpu.sync_copy(acc, num_out.at[hop].at[pl.ds(lo, rpt)])


def kernel(multi_input, edge_index_list, W_lin, b_lin, Wq, bq, Wk, bk, Wv, bv, Ws, bs):
    nhop, _, e = edge_index_list.shape
    n, d = multi_input.shape[1:]
    npad = -(-n // (BLK * NS)) * (BLK * NS)
    nch = -(-e // CI)
    nch += nch % 2
    epad = (nch + 1) * CI

    x = jnp.pad(multi_input, ((0, 0), (0, npad - n), (0, 0)))
    wqkv = jnp.concatenate([Wq, Wk, Wv], axis=2)
    bqkv = jnp.concatenate([bq, bk, bv], axis=1)[:, None, :]
    bb = (b_lin + bs[0] + bs[1])[None, :]
    ei = edge_index_list.astype(jnp.int32)
    erows = max(epad // 128, -(-(2 ** 21) // (nhop * 2 * 128)))
    srcs_p = jnp.pad(ei[:, 0], ((0, 0), (0, erows * 128 - e)))
    dsts_p = jnp.pad(ei[:, 1], ((0, 0), (0, erows * 128 - e)),
                     constant_values=npad)
    eipad = jnp.stack([srcs_p, dsts_p], axis=1).reshape(nhop, 2, erows, 128)

    grid = npad // BLK
    fvec = lambda: pl.BlockSpec((BLK, d), lambda i: (i, 0))
    proj = pl.pallas_call(
        _proj_body,
        grid=(grid,),
        in_specs=[
            fvec(), fvec(), fvec(),
            pl.BlockSpec((d, d), lambda i: (0, 0)),
            pl.BlockSpec((nhop, d, d), lambda i: (0, 0, 0)),
            pl.BlockSpec((nhop, d, 3 * d), lambda i: (0, 0, 0)),
            pl.BlockSpec((1, d), lambda i: (0, 0)),
            pl.BlockSpec((nhop, 1, 3 * d), lambda i: (0, 0, 0)),
        ],
        out_specs=[fvec(),
                   pl.BlockSpec((nhop, 3, BLK, d), lambda i: (0, 0, i, 0))],
        out_shape=[jax.ShapeDtypeStruct((npad, d), jnp.float32),
                   jax.ShapeDtypeStruct((nhop, 3, npad, d), jnp.float32)],
    )
    base, qkv = proj(x[0], x[1], x[2], W_lin, Ws, wqkv, bb, bqkv)

    mesh = plsc.VectorSubcoreMesh(core_axis_name="c", subcore_axis_name="s",
                                  num_cores=NCU)
    sc = pl.kernel(
        functools.partial(_sc_body, e, nch, npad, d),
        out_type=jax.ShapeDtypeStruct((nhop, npad, d + 128), jnp.float32),
        mesh=mesh,
        compiler_params=pltpu.CompilerParams(needs_layout_passes=False),
        scratch_types=[
            pltpu.VMEM((CROWS, 128), jnp.int32),
            pltpu.VMEM((CROWS, 128), jnp.int32),
            pltpu.VMEM((CROWS, 128), jnp.int32),
            pltpu.VMEM((CROWS, 128), jnp.int32),
            pltpu.VMEM((LN,), jnp.int32),
            pltpu.VMEM((LN,), jnp.int32),
            pltpu.VMEM((C,), jnp.int32),
            pltpu.VMEM((C,), jnp.int32),
            pltpu.VMEM((C, d), jnp.float32),
            pltpu.VMEM((C, d), jnp.float32),
            pltpu.VMEM((C, d), jnp.float32),
            pltpu.VMEM((C,), jnp.float32),
            pltpu.VMEM((npad // NW, d + 128), jnp.float32),
            pltpu.SMEM((1,), jnp.int32),
            pltpu.SemaphoreType.DMA,
            pltpu.SemaphoreType.DMA,
            pltpu.SemaphoreType.DMA,
            pltpu.SemaphoreType.DMA,
            pltpu.SemaphoreType.DMA,
            pltpu.SemaphoreType.DMA,
            pltpu.SemaphoreType.DMA,
        ],
    )
    agg = sc(qkv, eipad)

    post = pl.pallas_call(
        _post_body,
        grid=(grid,),
        in_specs=[
            fvec(),
            pl.BlockSpec((nhop, BLK, d + 128), lambda i: (0, i, 0)),
        ],
        out_specs=fvec(),
        out_shape=jax.ShapeDtypeStruct((npad, d), jnp.float32),
    )
    out = post(base, agg)
    return out[:n]


# speedup vs baseline: 2.6358x; 2.6358x over previous
import functools
import math

import jax
import jax.numpy as jnp
from jax import lax
from jax.experimental import pallas as pl
from jax.experimental.pallas import tpu as pltpu
from jax.experimental.pallas import tpu_sc as plsc

NCU = 2
NS = 16
NW = NCU * NS
LN = 16
C = 64
CI = 2048
BLK = 128


def _proj_body(x0, x1, x2, wlin, ws, wqkv, bb, bqkv, base, qkv):
    d = wlin.shape[0]
    acc = jnp.dot(x0[...], wlin[...], preferred_element_type=jnp.float32)
    acc += jnp.dot(x1[...], ws[0], preferred_element_type=jnp.float32)
    acc += jnp.dot(x2[...], ws[1], preferred_element_type=jnp.float32)
    base[...] = acc + bb[...]
    p0 = jnp.dot(x1[...], wqkv[0], preferred_element_type=jnp.float32) + bqkv[0]
    p1 = jnp.dot(x2[...], wqkv[1], preferred_element_type=jnp.float32) + bqkv[1]
    for j in range(3):
        qkv[0, j] = p0[:, j * d:(j + 1) * d]
        qkv[1, j] = p1[:, j * d:(j + 1) * d]


def _post_body(base, agg, out):
    s = agg[...]
    d = base.shape[1]
    r = base[...]
    for h in range(s.shape[0]):
        dd = jnp.sum(s[h][:, d:d + LN], axis=1, keepdims=True) + 1e-16
        r = r + s[h][:, :d] / dd
    out[...] = r


CROWS = CI // 128


def _sc_body(e_true, nch, nsp, d,
             qkv_hbm, ei_hbm, num_out,
             sa, da, sb, db, ria, rib, pend_src, pend_dst,
             q_rows, k_rows, v_rows, exbuf, acc, cnt,
             sem_sa, sem_da, sem_sb, sem_db, sem_q, sem_k, sem_v):
    cid = lax.axis_index("c")
    sid = lax.axis_index("s")
    gid = sid * NCU + cid
    ng = d // LN
    rpt = nsp // NW
    lo = gid * rpt
    hi = lo + rpt
    iota = lax.iota(jnp.int32, LN)
    zero = jnp.zeros((LN,), jnp.float32)
    zrow = iota * 0
    inv = 1.0 / math.sqrt(d)

    def pinit(i, cy):
        pend_src[pl.ds(i * LN, LN)] = zrow
        pend_dst[pl.ds(i * LN, LN)] = zrow + lo
        return cy
    lax.fori_loop(0, C // LN, pinit, 0)

    for hop in range(2):
        q_hbm = qkv_hbm.at[hop, 0]
        k_hbm = qkv_hbm.at[hop, 1]
        v_hbm = qkv_hbm.at[hop, 2]
        src_hbm = ei_hbm.at[hop, 0]
        dst_hbm = ei_hbm.at[hop, 1]

        def zacc(r, cy):
            for g in range(ng + 8):
                acc[r, pl.ds(g * LN, LN)] = zero
            return cy
        lax.fori_loop(0, rpt, zacc, 0)
        cnt[0] = 0

        def flush():
            cq = pltpu.async_copy(q_hbm.at[pend_dst], q_rows, sem_q)
            ck = pltpu.async_copy(k_hbm.at[pend_src], k_rows, sem_k)
            cv = pltpu.async_copy(v_hbm.at[pend_src], v_rows, sem_v)
            cq.wait()
            ck.wait()
            m = cnt[0]

            def dot_e(e, accs):
                p = q_rows[e, pl.ds(0, LN)] * k_rows[e, pl.ds(0, LN)]
                for dg in range(1, ng):
                    p += q_rows[e, pl.ds(dg * LN, LN)] * \
                        k_rows[e, pl.ds(dg * LN, LN)]
                accs = jnp.where(iota == (e & (LN - 1)), jnp.sum(p), accs)

                @pl.when((e & (LN - 1)) == LN - 1)
                def _():
                    gb = e - (LN - 1)
                    ex = jnp.where(gb + iota < m,
                                   jnp.exp(accs * inv), 0.0)
                    exbuf[pl.ds(gb, LN)] = ex
                return accs
            lax.fori_loop(0, C, dot_e, jnp.zeros((LN,), jnp.float32))
            cv.wait()

            def upd_g(g, cy):
                exv = exbuf[pl.ds(g * LN, LN)]
                dlv = pend_dst[pl.ds(g * LN, LN)] - lo
                for j in range(LN):
                    a = exv[j]
                    dl = dlv[j]
                    e = g * LN + j
                    row = zrow + dl
                    for dg in range(ng):
                        vv = v_rows[e, pl.ds(dg * LN, LN)] * a
                        plsc.addupdate_scatter(
                            acc, [row, dg * LN + iota], vv)
                    plsc.addupdate_scatter(
                        acc, [row, d + iota], jnp.where(iota == j, a, 0.0))
                return cy
            lax.fori_loop(0, C // LN, upd_g, 0)
            cnt[0] = 0

        def scan(sbuf, dbuf, base_pos):
            def grp(g, cy):
                r = g // 8
                cc = (g % 8) * LN
                srcg = sbuf[r, pl.ds(cc, LN)]
                dstg = dbuf[r, pl.ds(cc, LN)]
                match = (dstg >= lo) & (dstg < hi)

                @pl.when(cnt[0] >= C - LN)
                def _():
                    flush()

                off = cnt[0]
                plsc.store_compressed(pend_src.at[pl.ds(off, LN)], srcg,
                                      mask=match)
                plsc.store_compressed(pend_dst.at[pl.ds(off, LN)], dstg,
                                      mask=match)
                cnt[0] = off + plsc.all_reduce_population_count(match)[0]
                return cy
            lax.fori_loop(0, CI // LN, grp, 0)

        ria[pl.ds(0, LN)] = iota
        pltpu.async_copy(src_hbm.at[ria], sa, sem_sa)
        pltpu.async_copy(dst_hbm.at[ria], da, sem_da)

        def big(c2, cy):
            pltpu.make_async_copy(src_hbm.at[pl.ds(0, CROWS)], sa, sem_sa).wait()
            pltpu.make_async_copy(dst_hbm.at[pl.ds(0, CROWS)], da, sem_da).wait()
            rib[pl.ds(0, LN)] = (2 * c2 + 1) * CROWS + iota
            pltpu.async_copy(src_hbm.at[rib], sb, sem_sb)
            pltpu.async_copy(dst_hbm.at[rib], db, sem_db)
            scan(sa, da, 2 * c2 * CI)
            pltpu.make_async_copy(src_hbm.at[pl.ds(0, CROWS)], sb, sem_sb).wait()
            pltpu.make_async_copy(dst_hbm.at[pl.ds(0, CROWS)], db, sem_db).wait()
            ria[pl.ds(0, LN)] = (2 * c2 + 2) * CROWS + iota
            pltpu.async_copy(src_hbm.at[ria], sa, sem_sa)
            pltpu.async_copy(dst_hbm.at[ria], da, sem_da)
            scan(sb, db, (2 * c2 + 1) * CI)
            return cy
        lax.fori_loop(0, nch // 2, big, 0)
        pltpu.make_async_copy(src_hbm.at[pl.ds(0, CROWS)], sa, sem_sa).wait()
        pltpu.make_async_copy(dst_hbm.at[pl.ds(0, CROWS)], da, sem_da).wait()
        flush()

        pltpu.sync_copy(acc, num_out.at[hop].at[pl.ds(lo, rpt)])


def kernel(multi_input, edge_index_list, W_lin, b_lin, Wq, bq, Wk, bk, Wv, bv, Ws, bs):
    nhop, _, e = edge_index_list.shape
    n, d = multi_input.shape[1:]
    npad = -(-n // (BLK * NS)) * (BLK * NS)
    nch = -(-e // CI)
    nch += nch % 2
    epad = (nch + 1) * CI

    x = jnp.pad(multi_input, ((0, 0), (0, npad - n), (0, 0)))
    wqkv = jnp.concatenate([Wq, Wk, Wv], axis=2)
    bqkv = jnp.concatenate([bq, bk, bv], axis=1)[:, None, :]
    bb = (b_lin + bs[0] + bs[1])[None, :]
    ei = edge_index_list.astype(jnp.int32)
    erows = max(epad // 128, -(-(2 ** 21) // (nhop * 2 * 128)))
    srcs_p = jnp.pad(ei[:, 0], ((0, 0), (0, erows * 128 - e)))
    dsts_p = jnp.pad(ei[:, 1], ((0, 0), (0, erows * 128 - e)),
                     constant_values=npad)
    eipad = jnp.stack([srcs_p, dsts_p], axis=1).reshape(nhop, 2, erows, 128)

    grid = npad // BLK
    fvec = lambda: pl.BlockSpec((BLK, d), lambda i: (i, 0))
    proj = pl.pallas_call(
        _proj_body,
        grid=(grid,),
        in_specs=[
            fvec(), fvec(), fvec(),
            pl.BlockSpec((d, d), lambda i: (0, 0)),
            pl.BlockSpec((nhop, d, d), lambda i: (0, 0, 0)),
            pl.BlockSpec((nhop, d, 3 * d), lambda i: (0, 0, 0)),
            pl.BlockSpec((1, d), lambda i: (0, 0)),
            pl.BlockSpec((nhop, 1, 3 * d), lambda i: (0, 0, 0)),
        ],
        out_specs=[fvec(),
                   pl.BlockSpec((nhop, 3, BLK, d), lambda i: (0, 0, i, 0))],
        out_shape=[jax.ShapeDtypeStruct((npad, d), jnp.float32),
                   jax.ShapeDtypeStruct((nhop, 3, npad, d), jnp.float32)],
    )
    base, qkv = proj(x[0], x[1], x[2], W_lin, Ws, wqkv, bb, bqkv)

    mesh = plsc.VectorSubcoreMesh(core_axis_name="c", subcore_axis_name="s",
                                  num_cores=NCU)
    sc = pl.kernel(
        functools.partial(_sc_body, e, nch, npad, d),
        out_type=jax.ShapeDtypeStruct((nhop, npad, d + 128), jnp.float32),
        mesh=mesh,
        compiler_params=pltpu.CompilerParams(needs_layout_passes=False),
        scratch_types=[
            pltpu.VMEM((CROWS, 128), jnp.int32),
            pltpu.VMEM((CROWS, 128), jnp.int32),
            pltpu.VMEM((CROWS, 128), jnp.int32),
            pltpu.VMEM((CROWS, 128), jnp.int32),
            pltpu.VMEM((LN,), jnp.int32),
            pltpu.VMEM((LN,), jnp.int32),
            pltpu.VMEM((C,), jnp.int32),
            pltpu.VMEM((C,), jnp.int32),
            pltpu.VMEM((C, d), jnp.float32),
            pltpu.VMEM((C, d), jnp.float32),
            pltpu.VMEM((C, d), jnp.float32),
            pltpu.VMEM((C,), jnp.float32),
            pltpu.VMEM((npad // NW, d + 128), jnp.float32),
            pltpu.SMEM((1,), jnp.int32),
            pltpu.SemaphoreType.DMA,
            pltpu.SemaphoreType.DMA,
            pltpu.SemaphoreType.DMA,
            pltpu.SemaphoreType.DMA,
            pltpu.SemaphoreType.DMA,
            pltpu.SemaphoreType.DMA,
            pltpu.SemaphoreType.DMA,
        ],
    )
    agg = sc(qkv, eipad)

    post = pl.pallas_call(
        _post_body,
        grid=(grid,),
        in_specs=[
            fvec(),
            pl.BlockSpec((nhop, BLK, d + 128), lambda i: (0, i, 0)),
        ],
        out_specs=fvec(),
        out_shape=jax.ShapeDtypeStruct((npad, d), jnp.float32),
    )
    out = post(base, agg)
    return out[:n]


# speedup vs baseline: 2.6765x; 1.0154x over previous
import functools
import math

import jax
import jax.numpy as jnp
from jax import lax
from jax.experimental import pallas as pl
from jax.experimental.pallas import tpu as pltpu
from jax.experimental.pallas import tpu_sc as plsc

NCU = 2
NS = 16
NW = NCU * NS
LN = 16
C = 64
CI = 2048
BLK = 128


def _proj_body(x0, x1, x2, wlin, ws, wqkv, bb, bqkv, base, qkv):
    d = wlin.shape[0]
    acc = jnp.dot(x0[...], wlin[...], preferred_element_type=jnp.float32)
    acc += jnp.dot(x1[...], ws[0], preferred_element_type=jnp.float32)
    acc += jnp.dot(x2[...], ws[1], preferred_element_type=jnp.float32)
    base[...] = acc + bb[...]
    p0 = jnp.dot(x1[...], wqkv[0], preferred_element_type=jnp.float32) + bqkv[0]
    p1 = jnp.dot(x2[...], wqkv[1], preferred_element_type=jnp.float32) + bqkv[1]
    for j in range(3):
        qkv[0, j] = p0[:, j * d:(j + 1) * d]
        qkv[1, j] = p1[:, j * d:(j + 1) * d]


def _post_body(base, agg, out):
    s = agg[...]
    d = base.shape[1]
    r = base[...]
    for h in range(s.shape[0]):
        dd = jnp.sum(s[h][:, d:d + LN], axis=1, keepdims=True) + 1e-16
        r = r + s[h][:, :d] / dd
    out[...] = r


CROWS = CI // 128


def _sc_body(e_true, nch, nsp, d,
             qkv_hbm, ei_hbm, num_out,
             sa, da, sb, db, ria, rib, pend_src, pend_dst,
             q_rows, k_rows, v_rows, exbuf, acc, cnt,
             sem_sa, sem_da, sem_sb, sem_db, sem_q, sem_k, sem_v):
    cid = lax.axis_index("c")
    sid = lax.axis_index("s")
    gid = sid * NCU + cid
    ng = d // LN
    rpt = nsp // NW
    lo = gid * rpt
    hi = lo + rpt
    iota = lax.iota(jnp.int32, LN)
    zero = jnp.zeros((LN,), jnp.float32)
    zrow = iota * 0
    inv = 1.0 / math.sqrt(d)

    def pinit(i, cy):
        for p in range(2):
            pend_src[p, pl.ds(i * LN, LN)] = zrow
            pend_dst[p, pl.ds(i * LN, LN)] = zrow + lo
        return cy
    lax.fori_loop(0, C // LN, pinit, 0)

    for hop in range(2):
        q_hbm = qkv_hbm.at[hop, 0]
        k_hbm = qkv_hbm.at[hop, 1]
        v_hbm = qkv_hbm.at[hop, 2]
        src_hbm = ei_hbm.at[hop, 0]
        dst_hbm = ei_hbm.at[hop, 1]

        def zacc(r, cy):
            for g in range(ng + 8):
                acc[r, pl.ds(g * LN, LN)] = zero
            return cy
        lax.fori_loop(0, rpt, zacc, 0)
        cnt[0] = 0
        cnt[2] = 0
        cnt[3] = 0

        def process():
            pltpu.make_async_copy(q_hbm.at[pl.ds(0, C)], q_rows, sem_q).wait()
            pltpu.make_async_copy(q_hbm.at[pl.ds(0, C)], k_rows, sem_k).wait()
            m = cnt[1]
            parq = 1 - cnt[2]

            def dot_e(e, accs):
                p = q_rows[e, pl.ds(0, LN)] * k_rows[e, pl.ds(0, LN)]
                for dg in range(1, ng):
                    p += q_rows[e, pl.ds(dg * LN, LN)] * \
                        k_rows[e, pl.ds(dg * LN, LN)]
                accs = jnp.where(iota == (e & (LN - 1)), jnp.sum(p), accs)

                @pl.when((e & (LN - 1)) == LN - 1)
                def _():
                    gb = e - (LN - 1)
                    ex = jnp.where(gb + iota < m,
                                   jnp.exp(accs * inv), 0.0)
                    exbuf[pl.ds(gb, LN)] = ex
                return accs
            lax.fori_loop(0, C, dot_e, jnp.zeros((LN,), jnp.float32))
            pltpu.make_async_copy(q_hbm.at[pl.ds(0, C)], v_rows, sem_v).wait()

            def upd_g(g, cy):
                exv = exbuf[pl.ds(g * LN, LN)]
                dlv = pend_dst[parq, pl.ds(g * LN, LN)] - lo
                for j in range(LN):
                    a = exv[j]
                    dl = dlv[j]
                    e = g * LN + j
                    row = zrow + dl
                    for dg in range(ng):
                        vv = v_rows[e, pl.ds(dg * LN, LN)] * a
                        plsc.addupdate_scatter(
                            acc, [row, dg * LN + iota], vv)
                    plsc.addupdate_scatter(
                        acc, [row, d + iota], jnp.where(iota == j, a, 0.0))
                return cy
            lax.fori_loop(0, C // LN, upd_g, 0)

        def cycle(issue):
            @pl.when(cnt[3] > 0)
            def _():
                process()
            if issue:
                parf = cnt[2]
                pltpu.async_copy(q_hbm.at[pend_dst.at[parf]], q_rows, sem_q)
                pltpu.async_copy(k_hbm.at[pend_src.at[parf]], k_rows, sem_k)
                pltpu.async_copy(v_hbm.at[pend_src.at[parf]], v_rows, sem_v)
                cnt[1] = cnt[0]
                cnt[2] = 1 - parf
                cnt[0] = 0
                cnt[3] = 1
            else:
                cnt[3] = 0

        def scan(sbuf, dbuf, base_pos):
            def grp(g, cy):
                r = g // 8
                cc = (g % 8) * LN
                srcg = sbuf[r, pl.ds(cc, LN)]
                dstg = dbuf[r, pl.ds(cc, LN)]
                match = (dstg >= lo) & (dstg < hi)

                @pl.when(cnt[0] >= C - LN)
                def _():
                    cycle(True)

                off = cnt[0]
                parf = cnt[2]
                plsc.store_compressed(
                    pend_src.at[parf].at[pl.ds(off, LN)], srcg, mask=match)
                plsc.store_compressed(
                    pend_dst.at[parf].at[pl.ds(off, LN)], dstg, mask=match)
                cnt[0] = off + plsc.all_reduce_population_count(match)[0]
                return cy
            lax.fori_loop(0, CI // LN, grp, 0)

        ria[pl.ds(0, LN)] = iota
        pltpu.async_copy(src_hbm.at[ria], sa, sem_sa)
        pltpu.async_copy(dst_hbm.at[ria], da, sem_da)

        def big(c2, cy):
            pltpu.make_async_copy(src_hbm.at[pl.ds(0, CROWS)], sa, sem_sa).wait()
            pltpu.make_async_copy(dst_hbm.at[pl.ds(0, CROWS)], da, sem_da).wait()
            rib[pl.ds(0, LN)] = (2 * c2 + 1) * CROWS + iota
            pltpu.async_copy(src_hbm.at[rib], sb, sem_sb)
            pltpu.async_copy(dst_hbm.at[rib], db, sem_db)
            scan(sa, da, 2 * c2 * CI)
            pltpu.make_async_copy(src_hbm.at[pl.ds(0, CROWS)], sb, sem_sb).wait()
            pltpu.make_async_copy(dst_hbm.at[pl.ds(0, CROWS)], db, sem_db).wait()
            ria[pl.ds(0, LN)] = (2 * c2 + 2) * CROWS + iota
            pltpu.async_copy(src_hbm.at[ria], sa, sem_sa)
            pltpu.async_copy(dst_hbm.at[ria], da, sem_da)
            scan(sb, db, (2 * c2 + 1) * CI)
            return cy
        lax.fori_loop(0, nch // 2, big, 0)
        pltpu.make_async_copy(src_hbm.at[pl.ds(0, CROWS)], sa, sem_sa).wait()
        pltpu.make_async_copy(dst_hbm.at[pl.ds(0, CROWS)], da, sem_da).wait()
        cycle(True)
        cycle(False)

        pltpu.sync_copy(acc, num_out.at[hop].at[pl.ds(lo, rpt)])


def kernel(multi_input, edge_index_list, W_lin, b_lin, Wq, bq, Wk, bk, Wv, bv, Ws, bs):
    nhop, _, e = edge_index_list.shape
    n, d = multi_input.shape[1:]
    npad = -(-n // (BLK * NS)) * (BLK * NS)
    nch = -(-e // CI)
    nch += nch % 2
    epad = (nch + 1) * CI

    x = jnp.pad(multi_input, ((0, 0), (0, npad - n), (0, 0)))
    wqkv = jnp.concatenate([Wq, Wk, Wv], axis=2)
    bqkv = jnp.concatenate([bq, bk, bv], axis=1)[:, None, :]
    bb = (b_lin + bs[0] + bs[1])[None, :]
    ei = edge_index_list.astype(jnp.int32)
    erows = max(epad // 128, -(-(2 ** 21) // (nhop * 2 * 128)))
    srcs_p = jnp.pad(ei[:, 0], ((0, 0), (0, erows * 128 - e)))
    dsts_p = jnp.pad(ei[:, 1], ((0, 0), (0, erows * 128 - e)),
                     constant_values=npad)
    eipad = jnp.stack([srcs_p, dsts_p], axis=1).reshape(nhop, 2, erows, 128)

    grid = npad // BLK
    fvec = lambda: pl.BlockSpec((BLK, d), lambda i: (i, 0))
    proj = pl.pallas_call(
        _proj_body,
        grid=(grid,),
        in_specs=[
            fvec(), fvec(), fvec(),
            pl.BlockSpec((d, d), lambda i: (0, 0)),
            pl.BlockSpec((nhop, d, d), lambda i: (0, 0, 0)),
            pl.BlockSpec((nhop, d, 3 * d), lambda i: (0, 0, 0)),
            pl.BlockSpec((1, d), lambda i: (0, 0)),
            pl.BlockSpec((nhop, 1, 3 * d), lambda i: (0, 0, 0)),
        ],
        out_specs=[fvec(),
                   pl.BlockSpec((nhop, 3, BLK, d), lambda i: (0, 0, i, 0))],
        out_shape=[jax.ShapeDtypeStruct((npad, d), jnp.float32),
                   jax.ShapeDtypeStruct((nhop, 3, npad, d), jnp.float32)],
    )
    base, qkv = proj(x[0], x[1], x[2], W_lin, Ws, wqkv, bb, bqkv)

    mesh = plsc.VectorSubcoreMesh(core_axis_name="c", subcore_axis_name="s",
                                  num_cores=NCU)
    sc = pl.kernel(
        functools.partial(_sc_body, e, nch, npad, d),
        out_type=jax.ShapeDtypeStruct((nhop, npad, d + 128), jnp.float32),
        mesh=mesh,
        compiler_params=pltpu.CompilerParams(needs_layout_passes=False),
        scratch_types=[
            pltpu.VMEM((CROWS, 128), jnp.int32),
            pltpu.VMEM((CROWS, 128), jnp.int32),
            pltpu.VMEM((CROWS, 128), jnp.int32),
            pltpu.VMEM((CROWS, 128), jnp.int32),
            pltpu.VMEM((LN,), jnp.int32),
            pltpu.VMEM((LN,), jnp.int32),
            pltpu.VMEM((2, C), jnp.int32),
            pltpu.VMEM((2, C), jnp.int32),
            pltpu.VMEM((C, d), jnp.float32),
            pltpu.VMEM((C, d), jnp.float32),
            pltpu.VMEM((C, d), jnp.float32),
            pltpu.VMEM((C,), jnp.float32),
            pltpu.VMEM((npad // NW, d + 128), jnp.float32),
            pltpu.SMEM((4,), jnp.int32),
            pltpu.SemaphoreType.DMA,
            pltpu.SemaphoreType.DMA,
            pltpu.SemaphoreType.DMA,
            pltpu.SemaphoreType.DMA,
            pltpu.SemaphoreType.DMA,
            pltpu.SemaphoreType.DMA,
            pltpu.SemaphoreType.DMA,
        ],
    )
    agg = sc(qkv, eipad)

    post = pl.pallas_call(
        _post_body,
        grid=(grid,),
        in_specs=[
            fvec(),
            pl.BlockSpec((nhop, BLK, d + 128), lambda i: (0, i, 0)),
        ],
        out_specs=fvec(),
        out_shape=jax.ShapeDtypeStruct((npad, d), jnp.float32),
    )
    out = post(base, agg)
    return out[:n]


# speedup vs baseline: 3.3440x; 1.2494x over previous
import functools
import math

import jax
import jax.numpy as jnp
from jax import lax
from jax.experimental import pallas as pl
from jax.experimental.pallas import tpu as pltpu
from jax.experimental.pallas import tpu_sc as plsc

NCU = 2
NS = 16
NW = NCU * NS
LN = 16
C = 80
CI = 2048
BLK = 128


def _proj_body(x0, x1, x2, wlin, ws, wqkv, bb, bqkv, base, qkv):
    d = wlin.shape[0]
    acc = jnp.dot(x0[...], wlin[...], preferred_element_type=jnp.float32)
    acc += jnp.dot(x1[...], ws[0], preferred_element_type=jnp.float32)
    acc += jnp.dot(x2[...], ws[1], preferred_element_type=jnp.float32)
    base[...] = acc + bb[...]
    p0 = jnp.dot(x1[...], wqkv[0], preferred_element_type=jnp.float32) + bqkv[0]
    p1 = jnp.dot(x2[...], wqkv[1], preferred_element_type=jnp.float32) + bqkv[1]
    for j in range(3):
        qkv[0, j] = p0[:, j * d:(j + 1) * d]
        qkv[1, j] = p1[:, j * d:(j + 1) * d]


def _post_body(base, agg, out):
    s = agg[...]
    d = base.shape[1]
    r = base[...]
    for h in range(s.shape[0]):
        dd = jnp.sum(s[h][:, d:d + LN], axis=1, keepdims=True) + 1e-16
        r = r + s[h][:, :d] / dd
    out[...] = r


CROWS = CI // 128


def _sc_body(e_true, nch, nsp, d,
             qkv_hbm, ei_hbm, num_out,
             sa, da, sb, db, ria, rib, pend_src, pend_dst,
             q_rows, k_rows, v_rows, exbuf, acc, cnt,
             sem_sa, sem_da, sem_sb, sem_db, sem_q, sem_k, sem_v):
    cid = lax.axis_index("c")
    sid = lax.axis_index("s")
    gid = sid * NCU + cid
    ng = d // LN
    rpt = nsp // NW
    lo = gid * rpt
    hi = lo + rpt
    iota = lax.iota(jnp.int32, LN)
    zero = jnp.zeros((LN,), jnp.float32)
    zrow = iota * 0
    inv = 1.0 / math.sqrt(d)

    def pinit(i, cy):
        for p in range(2):
            pend_src[p, pl.ds(i * LN, LN)] = zrow
            pend_dst[p, pl.ds(i * LN, LN)] = zrow + lo
        return cy
    lax.fori_loop(0, C // LN, pinit, 0)

    for hop in range(2):
        q_hbm = qkv_hbm.at[hop, 0]
        k_hbm = qkv_hbm.at[hop, 1]
        v_hbm = qkv_hbm.at[hop, 2]
        src_hbm = ei_hbm.at[hop, 0]
        dst_hbm = ei_hbm.at[hop, 1]

        def zacc(r, cy):
            for g in range(ng + 8):
                acc[r, pl.ds(g * LN, LN)] = zero
            return cy
        lax.fori_loop(0, rpt, zacc, 0)
        cnt[0] = 0
        cnt[2] = 0
        cnt[3] = 0

        def process():
            pltpu.make_async_copy(q_hbm.at[pl.ds(0, C)], q_rows, sem_q).wait()
            pltpu.make_async_copy(q_hbm.at[pl.ds(0, C)], k_rows, sem_k).wait()
            m = cnt[1]
            parq = 1 - cnt[2]

            def dot_e(e, accs):
                p = q_rows[e, pl.ds(0, LN)] * k_rows[e, pl.ds(0, LN)]
                for dg in range(1, ng):
                    p += q_rows[e, pl.ds(dg * LN, LN)] * \
                        k_rows[e, pl.ds(dg * LN, LN)]
                accs = jnp.where(iota == (e & (LN - 1)), jnp.sum(p), accs)

                @pl.when((e & (LN - 1)) == LN - 1)
                def _():
                    gb = e - (LN - 1)
                    ex = jnp.where(gb + iota < m,
                                   jnp.exp(accs * inv), 0.0)
                    exbuf[pl.ds(gb, LN)] = ex
                return accs
            lax.fori_loop(0, C, dot_e, jnp.zeros((LN,), jnp.float32))
            pltpu.make_async_copy(q_hbm.at[pl.ds(0, C)], v_rows, sem_v).wait()

            def upd_g(g, cy):
                exv = exbuf[pl.ds(g * LN, LN)]
                dlv = pend_dst[parq, pl.ds(g * LN, LN)] - lo
                for j in range(LN):
                    a = exv[j]
                    dl = dlv[j]
                    e = g * LN + j
                    row = zrow + dl
                    for dg in range(ng):
                        vv = v_rows[e, pl.ds(dg * LN, LN)] * a
                        plsc.addupdate_scatter(
                            acc, [row, dg * LN + iota], vv)
                    plsc.addupdate_scatter(
                        acc, [row, d + iota], jnp.where(iota == j, a, 0.0))
                return cy
            lax.fori_loop(0, C // LN, upd_g, 0)

        def cycle(issue):
            @pl.when(cnt[3] > 0)
            def _():
                process()
            if issue:
                parf = cnt[2]
                pltpu.async_copy(q_hbm.at[pend_dst.at[parf]], q_rows, sem_q)
                pltpu.async_copy(k_hbm.at[pend_src.at[parf]], k_rows, sem_k)
                pltpu.async_copy(v_hbm.at[pend_src.at[parf]], v_rows, sem_v)
                cnt[1] = cnt[0]
                cnt[2] = 1 - parf
                cnt[0] = 0
                cnt[3] = 1
            else:
                cnt[3] = 0

        def scan(sbuf, dbuf, base_pos):
            def grp(g, cy):
                r = g // 8
                cc = (g % 8) * LN
                srcg = sbuf[r, pl.ds(cc, LN)]
                dstg = dbuf[r, pl.ds(cc, LN)]
                match = (dstg >= lo) & (dstg < hi)

                @pl.when(cnt[0] >= C - LN)
                def _():
                    cycle(True)

                off = cnt[0]
                parf = cnt[2]
                plsc.store_compressed(
                    pend_src.at[parf].at[pl.ds(off, LN)], srcg, mask=match)
                plsc.store_compressed(
                    pend_dst.at[parf].at[pl.ds(off, LN)], dstg, mask=match)
                cnt[0] = off + plsc.all_reduce_population_count(match)[0]
                return cy
            lax.fori_loop(0, CI // LN, grp, 0)

        ria[pl.ds(0, LN)] = iota
        pltpu.async_copy(src_hbm.at[ria], sa, sem_sa)
        pltpu.async_copy(dst_hbm.at[ria], da, sem_da)

        def big(c2, cy):
            pltpu.make_async_copy(src_hbm.at[pl.ds(0, CROWS)], sa, sem_sa).wait()
            pltpu.make_async_copy(dst_hbm.at[pl.ds(0, CROWS)], da, sem_da).wait()
            rib[pl.ds(0, LN)] = (2 * c2 + 1) * CROWS + iota
            pltpu.async_copy(src_hbm.at[rib], sb, sem_sb)
            pltpu.async_copy(dst_hbm.at[rib], db, sem_db)
            scan(sa, da, 2 * c2 * CI)
            pltpu.make_async_copy(src_hbm.at[pl.ds(0, CROWS)], sb, sem_sb).wait()
            pltpu.make_async_copy(dst_hbm.at[pl.ds(0, CROWS)], db, sem_db).wait()
            ria[pl.ds(0, LN)] = (2 * c2 + 2) * CROWS + iota
            pltpu.async_copy(src_hbm.at[ria], sa, sem_sa)
            pltpu.async_copy(dst_hbm.at[ria], da, sem_da)
            scan(sb, db, (2 * c2 + 1) * CI)
            return cy
        lax.fori_loop(0, nch // 2, big, 0)
        pltpu.make_async_copy(src_hbm.at[pl.ds(0, CROWS)], sa, sem_sa).wait()
        pltpu.make_async_copy(dst_hbm.at[pl.ds(0, CROWS)], da, sem_da).wait()
        cycle(True)
        cycle(False)

        pltpu.sync_copy(acc, num_out.at[hop].at[pl.ds(lo, rpt)])


def kernel(multi_input, edge_index_list, W_lin, b_lin, Wq, bq, Wk, bk, Wv, bv, Ws, bs):
    nhop, _, e = edge_index_list.shape
    n, d = multi_input.shape[1:]
    npad = -(-n // (BLK * NS)) * (BLK * NS)
    nch = -(-e // CI)
    nch += nch % 2
    epad = (nch + 1) * CI

    x = jnp.pad(multi_input, ((0, 0), (0, npad - n), (0, 0)))
    wqkv = jnp.concatenate([Wq, Wk, Wv], axis=2)
    bqkv = jnp.concatenate([bq, bk, bv], axis=1)[:, None, :]
    bb = (b_lin + bs[0] + bs[1])[None, :]
    ei = edge_index_list.astype(jnp.int32)
    erows = max(epad // 128, -(-(2 ** 21) // (nhop * 2 * 128)))
    srcs_p = jnp.pad(ei[:, 0], ((0, 0), (0, erows * 128 - e)))
    dsts_p = jnp.pad(ei[:, 1], ((0, 0), (0, erows * 128 - e)),
                     constant_values=npad)
    eipad = jnp.stack([srcs_p, dsts_p], axis=1).reshape(nhop, 2, erows, 128)

    grid = npad // BLK
    fvec = lambda: pl.BlockSpec((BLK, d), lambda i: (i, 0))
    proj = pl.pallas_call(
        _proj_body,
        grid=(grid,),
        in_specs=[
            fvec(), fvec(), fvec(),
            pl.BlockSpec((d, d), lambda i: (0, 0)),
            pl.BlockSpec((nhop, d, d), lambda i: (0, 0, 0)),
            pl.BlockSpec((nhop, d, 3 * d), lambda i: (0, 0, 0)),
            pl.BlockSpec((1, d), lambda i: (0, 0)),
            pl.BlockSpec((nhop, 1, 3 * d), lambda i: (0, 0, 0)),
        ],
        out_specs=[fvec(),
                   pl.BlockSpec((nhop, 3, BLK, d), lambda i: (0, 0, i, 0))],
        out_shape=[jax.ShapeDtypeStruct((npad, d), jnp.float32),
                   jax.ShapeDtypeStruct((nhop, 3, npad, d), jnp.float32)],
    )
    base, qkv = proj(x[0], x[1], x[2], W_lin, Ws, wqkv, bb, bqkv)

    mesh = plsc.VectorSubcoreMesh(core_axis_name="c", subcore_axis_name="s",
                                  num_cores=NCU)
    sc = pl.kernel(
        functools.partial(_sc_body, e, nch, npad, d),
        out_type=jax.ShapeDtypeStruct((nhop, npad, d + 128), jnp.float32),
        mesh=mesh,
        compiler_params=pltpu.CompilerParams(needs_layout_passes=False),
        scratch_types=[
            pltpu.VMEM((CROWS, 128), jnp.int32),
            pltpu.VMEM((CROWS, 128), jnp.int32),
            pltpu.VMEM((CROWS, 128), jnp.int32),
            pltpu.VMEM((CROWS, 128), jnp.int32),
            pltpu.VMEM((LN,), jnp.int32),
            pltpu.VMEM((LN,), jnp.int32),
            pltpu.VMEM((2, C), jnp.int32),
            pltpu.VMEM((2, C), jnp.int32),
            pltpu.VMEM((C, d), jnp.float32),
            pltpu.VMEM((C, d), jnp.float32),
            pltpu.VMEM((C, d), jnp.float32),
            pltpu.VMEM((C,), jnp.float32),
            pltpu.VMEM((npad // NW, d + 128), jnp.float32),
            pltpu.SMEM((4,), jnp.int32),
            pltpu.SemaphoreType.DMA,
            pltpu.SemaphoreType.DMA,
            pltpu.SemaphoreType.DMA,
            pltpu.SemaphoreType.DMA,
            pltpu.SemaphoreType.DMA,
            pltpu.SemaphoreType.DMA,
            pltpu.SemaphoreType.DMA,
        ],
    )
    agg = sc(qkv, eipad)

    post = pl.pallas_call(
        _post_body,
        grid=(grid,),
        in_specs=[
            fvec(),
            pl.BlockSpec((nhop, BLK, d + 128), lambda i: (0, i, 0)),
        ],
        out_specs=fvec(),
        out_shape=jax.ShapeDtypeStruct((npad, d), jnp.float32),
    )
    out = post(base, agg)
    return out[:n]


# speedup vs baseline: 3.9843x; 1.1915x over previous
import functools
import math

import jax
import jax.numpy as jnp
from jax import lax
from jax.experimental import pallas as pl
from jax.experimental.pallas import tpu as pltpu
from jax.experimental.pallas import tpu_sc as plsc

NCU = 2
NS = 16
NW = NCU * NS
LN = 16
C = 96
CI = 2048
BLK = 128


def _proj_body(x0, x1, x2, wlin, ws, wqkv, bb, bqkv, base, qkv):
    d = wlin.shape[0]
    acc = jnp.dot(x0[...], wlin[...], preferred_element_type=jnp.float32)
    acc += jnp.dot(x1[...], ws[0], preferred_element_type=jnp.float32)
    acc += jnp.dot(x2[...], ws[1], preferred_element_type=jnp.float32)
    base[...] = acc + bb[...]
    p0 = jnp.dot(x1[...], wqkv[0], preferred_element_type=jnp.float32) + bqkv[0]
    p1 = jnp.dot(x2[...], wqkv[1], preferred_element_type=jnp.float32) + bqkv[1]
    for j in range(3):
        qkv[0, j] = p0[:, j * d:(j + 1) * d]
        qkv[1, j] = p1[:, j * d:(j + 1) * d]


def _post_body(base, agg, out):
    s = agg[...]
    d = base.shape[1]
    r = base[...]
    for h in range(s.shape[0]):
        dd = jnp.sum(s[h][:, d:d + LN], axis=1, keepdims=True) + 1e-16
        r = r + s[h][:, :d] / dd
    out[...] = r


CROWS = CI // 128


def _sc_body(e_true, nch, nsp, d,
             qkv_hbm, ei_hbm, num_out,
             sa, da, sb, db, ria, rib, pend_src, pend_dst,
             q_rows, k_rows, v_rows, exbuf, acc, cnt,
             sem_sa, sem_da, sem_sb, sem_db, sem_q, sem_k, sem_v):
    cid = lax.axis_index("c")
    sid = lax.axis_index("s")
    gid = sid * NCU + cid
    ng = d // LN
    rpt = nsp // NW
    lo = gid * rpt
    hi = lo + rpt
    iota = lax.iota(jnp.int32, LN)
    zero = jnp.zeros((LN,), jnp.float32)
    zrow = iota * 0
    inv = 1.0 / math.sqrt(d)

    def pinit(i, cy):
        for p in range(2):
            pend_src[p, pl.ds(i * LN, LN)] = zrow
            pend_dst[p, pl.ds(i * LN, LN)] = zrow + lo
        return cy
    lax.fori_loop(0, C // LN, pinit, 0)

    for hop in range(2):
        q_hbm = qkv_hbm.at[hop, 0]
        k_hbm = qkv_hbm.at[hop, 1]
        v_hbm = qkv_hbm.at[hop, 2]
        src_hbm = ei_hbm.at[hop, 0]
        dst_hbm = ei_hbm.at[hop, 1]

        def zacc(r, cy):
            for g in range(ng + 8):
                acc[r, pl.ds(g * LN, LN)] = zero
            return cy
        lax.fori_loop(0, rpt, zacc, 0)
        cnt[0] = 0
        cnt[2] = 0
        cnt[3] = 0

        def process():
            pltpu.make_async_copy(q_hbm.at[pl.ds(0, C)], q_rows, sem_q).wait()
            pltpu.make_async_copy(q_hbm.at[pl.ds(0, C)], k_rows, sem_k).wait()
            m = cnt[1]
            parq = 1 - cnt[2]

            def dot_e(e, accs):
                p = q_rows[e, pl.ds(0, LN)] * k_rows[e, pl.ds(0, LN)]
                for dg in range(1, ng):
                    p += q_rows[e, pl.ds(dg * LN, LN)] * \
                        k_rows[e, pl.ds(dg * LN, LN)]
                accs = jnp.where(iota == (e & (LN - 1)), jnp.sum(p), accs)

                @pl.when((e & (LN - 1)) == LN - 1)
                def _():
                    gb = e - (LN - 1)
                    ex = jnp.where(gb + iota < m,
                                   jnp.exp(accs * inv), 0.0)
                    exbuf[pl.ds(gb, LN)] = ex
                return accs
            lax.fori_loop(0, C, dot_e, jnp.zeros((LN,), jnp.float32))
            pltpu.make_async_copy(q_hbm.at[pl.ds(0, C)], v_rows, sem_v).wait()

            def upd_g(g, cy):
                exv = exbuf[pl.ds(g * LN, LN)]
                dlv = pend_dst[parq, pl.ds(g * LN, LN)] - lo
                for j in range(LN):
                    a = exv[j]
                    dl = dlv[j]
                    e = g * LN + j
                    row = zrow + dl
                    for dg in range(ng):
                        vv = v_rows[e, pl.ds(dg * LN, LN)] * a
                        plsc.addupdate_scatter(
                            acc, [row, dg * LN + iota], vv)
                    plsc.addupdate_scatter(
                        acc, [row, d + iota], jnp.where(iota == j, a, 0.0))
                return cy
            lax.fori_loop(0, C // LN, upd_g, 0)

        def cycle(issue):
            @pl.when(cnt[3] > 0)
            def _():
                process()
            if issue:
                parf = cnt[2]
                pltpu.async_copy(q_hbm.at[pend_dst.at[parf]], q_rows, sem_q)
                pltpu.async_copy(k_hbm.at[pend_src.at[parf]], k_rows, sem_k)
                pltpu.async_copy(v_hbm.at[pend_src.at[parf]], v_rows, sem_v)
                cnt[1] = cnt[0]
                cnt[2] = 1 - parf
                cnt[0] = 0
                cnt[3] = 1
            else:
                cnt[3] = 0

        def scan(sbuf, dbuf, base_pos):
            def grp(g, cy):
                r = g // 8
                cc = (g % 8) * LN
                srcg = sbuf[r, pl.ds(cc, LN)]
                dstg = dbuf[r, pl.ds(cc, LN)]
                match = (dstg >= lo) & (dstg < hi)

                @pl.when(cnt[0] >= C - LN)
                def _():
                    cycle(True)

                off = cnt[0]
                parf = cnt[2]
                plsc.store_compressed(
                    pend_src.at[parf].at[pl.ds(off, LN)], srcg, mask=match)
                plsc.store_compressed(
                    pend_dst.at[parf].at[pl.ds(off, LN)], dstg, mask=match)
                cnt[0] = off + plsc.all_reduce_population_count(match)[0]
                return cy
            lax.fori_loop(0, CI // LN, grp, 0)

        ria[pl.ds(0, LN)] = iota
        pltpu.async_copy(src_hbm.at[ria], sa, sem_sa)
        pltpu.async_copy(dst_hbm.at[ria], da, sem_da)

        def big(c2, cy):
            pltpu.make_async_copy(src_hbm.at[pl.ds(0, CROWS)], sa, sem_sa).wait()
            pltpu.make_async_copy(dst_hbm.at[pl.ds(0, CROWS)], da, sem_da).wait()
            rib[pl.ds(0, LN)] = (2 * c2 + 1) * CROWS + iota
            pltpu.async_copy(src_hbm.at[rib], sb, sem_sb)
            pltpu.async_copy(dst_hbm.at[rib], db, sem_db)
            scan(sa, da, 2 * c2 * CI)
            pltpu.make_async_copy(src_hbm.at[pl.ds(0, CROWS)], sb, sem_sb).wait()
            pltpu.make_async_copy(dst_hbm.at[pl.ds(0, CROWS)], db, sem_db).wait()
            ria[pl.ds(0, LN)] = (2 * c2 + 2) * CROWS + iota
            pltpu.async_copy(src_hbm.at[ria], sa, sem_sa)
            pltpu.async_copy(dst_hbm.at[ria], da, sem_da)
            scan(sb, db, (2 * c2 + 1) * CI)
            return cy
        lax.fori_loop(0, nch // 2, big, 0)
        pltpu.make_async_copy(src_hbm.at[pl.ds(0, CROWS)], sa, sem_sa).wait()
        pltpu.make_async_copy(dst_hbm.at[pl.ds(0, CROWS)], da, sem_da).wait()
        cycle(True)
        cycle(False)

        pltpu.sync_copy(acc, num_out.at[hop].at[pl.ds(lo, rpt)])


def kernel(multi_input, edge_index_list, W_lin, b_lin, Wq, bq, Wk, bk, Wv, bv, Ws, bs):
    nhop, _, e = edge_index_list.shape
    n, d = multi_input.shape[1:]
    npad = -(-n // (BLK * NS)) * (BLK * NS)
    nch = -(-e // CI)
    nch += nch % 2
    epad = (nch + 1) * CI

    x = jnp.pad(multi_input, ((0, 0), (0, npad - n), (0, 0)))
    wqkv = jnp.concatenate([Wq, Wk, Wv], axis=2)
    bqkv = jnp.concatenate([bq, bk, bv], axis=1)[:, None, :]
    bb = (b_lin + bs[0] + bs[1])[None, :]
    ei = edge_index_list.astype(jnp.int32)
    erows = max(epad // 128, -(-(2 ** 21) // (nhop * 2 * 128)))
    srcs_p = jnp.pad(ei[:, 0], ((0, 0), (0, erows * 128 - e)))
    dsts_p = jnp.pad(ei[:, 1], ((0, 0), (0, erows * 128 - e)),
                     constant_values=npad)
    eipad = jnp.stack([srcs_p, dsts_p], axis=1).reshape(nhop, 2, erows, 128)

    grid = npad // BLK
    fvec = lambda: pl.BlockSpec((BLK, d), lambda i: (i, 0))
    proj = pl.pallas_call(
        _proj_body,
        grid=(grid,),
        in_specs=[
            fvec(), fvec(), fvec(),
            pl.BlockSpec((d, d), lambda i: (0, 0)),
            pl.BlockSpec((nhop, d, d), lambda i: (0, 0, 0)),
            pl.BlockSpec((nhop, d, 3 * d), lambda i: (0, 0, 0)),
            pl.BlockSpec((1, d), lambda i: (0, 0)),
            pl.BlockSpec((nhop, 1, 3 * d), lambda i: (0, 0, 0)),
        ],
        out_specs=[fvec(),
                   pl.BlockSpec((nhop, 3, BLK, d), lambda i: (0, 0, i, 0))],
        out_shape=[jax.ShapeDtypeStruct((npad, d), jnp.float32),
                   jax.ShapeDtypeStruct((nhop, 3, npad, d), jnp.float32)],
    )
    base, qkv = proj(x[0], x[1], x[2], W_lin, Ws, wqkv, bb, bqkv)

    mesh = plsc.VectorSubcoreMesh(core_axis_name="c", subcore_axis_name="s",
                                  num_cores=NCU)
    sc = pl.kernel(
        functools.partial(_sc_body, e, nch, npad, d),
        out_type=jax.ShapeDtypeStruct((nhop, npad, d + 128), jnp.float32),
        mesh=mesh,
        compiler_params=pltpu.CompilerParams(needs_layout_passes=False),
        scratch_types=[
            pltpu.VMEM((CROWS, 128), jnp.int32),
            pltpu.VMEM((CROWS, 128), jnp.int32),
            pltpu.VMEM((CROWS, 128), jnp.int32),
            pltpu.VMEM((CROWS, 128), jnp.int32),
            pltpu.VMEM((LN,), jnp.int32),
            pltpu.VMEM((LN,), jnp.int32),
            pltpu.VMEM((2, C), jnp.int32),
            pltpu.VMEM((2, C), jnp.int32),
            pltpu.VMEM((C, d), jnp.float32),
            pltpu.VMEM((C, d), jnp.float32),
            pltpu.VMEM((C, d), jnp.float32),
            pltpu.VMEM((C,), jnp.float32),
            pltpu.VMEM((npad // NW, d + 128), jnp.float32),
            pltpu.SMEM((4,), jnp.int32),
            pltpu.SemaphoreType.DMA,
            pltpu.SemaphoreType.DMA,
            pltpu.SemaphoreType.DMA,
            pltpu.SemaphoreType.DMA,
            pltpu.SemaphoreType.DMA,
            pltpu.SemaphoreType.DMA,
            pltpu.SemaphoreType.DMA,
        ],
    )
    agg = sc(qkv, eipad)

    post = pl.pallas_call(
        _post_body,
        grid=(grid,),
        in_specs=[
            fvec(),
            pl.BlockSpec((nhop, BLK, d + 128), lambda i: (0, i, 0)),
        ],
        out_specs=fvec(),
        out_shape=jax.ShapeDtypeStruct((npad, d), jnp.float32),
    )
    out = post(base, agg)
    return out[:n]
